# baseline (speedup 1.0000x reference)
import jax
import jax.numpy as jnp
from jax import lax
from jax.experimental import pallas as pl
from jax.experimental.pallas import tpu as pltpu

B = 1024
D = 128
C = 128
H = 8
W = 8
S = H * W
P = 64
BB = 128
NB = B // BB
K3 = 3 * P


def _body(fm_ref, cx_ref, cy_ref, lab_ref, emb_ref, meansT_ref, me_ref,
          dist_ref, val_ref, loss_ref,
          same_scr, accd_scr, accs_scr):
    pid = pl.program_id(0)

    @pl.when(pid < NB)
    def _main():
        e = emb_ref[...]
        d2s = []
        for k in range(3):
            de = e - me_ref[k:k + 1, :]
            d2s.append(jnp.sum(de * de, axis=1, keepdims=True))
        dist = jnp.sqrt(jnp.concatenate(d2s, axis=1))
        dist_ref[...] = dist
        da = dist[:, 0:1]
        db = dist[:, 1:2]
        dc = dist[:, 2:3]
        val = jnp.where((da <= db) & (da <= dc), 0,
                        jnp.where(db <= dc, 1, 2)).astype(jnp.int32)
        val_ref[...] = val

        cx = cx_ref[...]
        cy = cy_ref[...]
        pos_t = ((cx * 2341) >> 16) * W + ((cy * 2341) >> 16)
        pos = jnp.transpose(pos_t)
        lab = jnp.broadcast_to(
            jnp.transpose(lab_ref[0]), (BB, P))

        oh = (lax.broadcasted_iota(jnp.int32, (BB, S, P), 1)
              == pos[:, None, :]).astype(jnp.float32)
        fm = fm_ref[...]
        peT = lax.dot_general(fm, oh, (((1,), (1,)), ((0,), (0,))),
                              preferred_element_type=jnp.float32)
        meansT = meansT_ref[...]
        ones_c = jnp.ones((C,), jnp.float32)

        def _csum(x):
            return lax.dot_general(x, ones_c, (((1,), (0,)), ((), ())),
                                   preferred_element_type=jnp.float32)

        pe_norm = _csum(peT * peT)
        m_norm = jnp.sum(meansT * meansT, axis=0, keepdims=True)
        dks = []
        for k in range(3):
            mk = meansT[None, :, k * P:(k + 1) * P]
            cross = _csum(peT * mk)
            d2 = pe_norm - 2.0 * cross + m_norm[:, k * P:(k + 1) * P]
            dks.append(jnp.sqrt(jnp.maximum(d2, 0.0)))
        same = jnp.where(lab == 0, dks[0],
                         jnp.where(lab == 1, dks[1], dks[2]))
        same_scr[pl.ds(pid * BB, BB), :] = same

        pd = jnp.sum(dks[0] + dks[1] + dks[2], axis=0, keepdims=True)
        ps = jnp.sum(same, axis=0, keepdims=True)

        @pl.when(pid == 0)
        def _init():
            accd_scr[...] = pd
            accs_scr[...] = ps

        @pl.when(pid > 0)
        def _acc():
            accd_scr[...] += pd
            accs_scr[...] += ps

    @pl.when(pid == NB)
    def _loss():
        sum_dis = jnp.sum(accd_scr[...])
        sum_same = jnp.sum(accs_scr[...])
        diff_mean = (sum_dis - sum_same) * (1.0 / (B * 2 * P))
        s = same_scr[...]
        t = jnp.maximum(s + (1.0 - diff_mean), 0.0)
        loss_ref[...] = (jnp.sum(t) * (1.0 / (B * P))).reshape(1, 1)


def kernel(labels, embeddings, feature_maps, means_b, means_m, means_n,
           centroids_x, centroids_y, mean_embedding_b, mean_embedding_m,
           mean_embedding_n):
    fm = jnp.transpose(feature_maps, (0, 2, 3, 1)).reshape(B, S, C)
    lab3 = labels.astype(jnp.int32).reshape(NB, 1, BB)
    meansT = jnp.concatenate([means_b.T, means_m.T, means_n.T], axis=1)
    me = jnp.stack([mean_embedding_b, mean_embedding_m, mean_embedding_n])

    clamp = NB - 1
    dist, val, loss = pl.pallas_call(
        _body,
        grid=(NB + 1,),
        in_specs=[
            pl.BlockSpec((BB, S, C), lambda i: (jnp.minimum(i, clamp), 0, 0)),
            pl.BlockSpec((P, BB), lambda i: (0, jnp.minimum(i, clamp))),
            pl.BlockSpec((P, BB), lambda i: (0, jnp.minimum(i, clamp))),
            pl.BlockSpec((1, 1, BB), lambda i: (jnp.minimum(i, clamp), 0, 0)),
            pl.BlockSpec((BB, D), lambda i: (jnp.minimum(i, clamp), 0)),
            pl.BlockSpec((C, K3), lambda i: (0, 0)),
            pl.BlockSpec((3, D), lambda i: (0, 0)),
        ],
        out_specs=[
            pl.BlockSpec((BB, 3), lambda i: (jnp.minimum(i, clamp), 0)),
            pl.BlockSpec((BB, 1), lambda i: (jnp.minimum(i, clamp), 0)),
            pl.BlockSpec((1, 1), lambda i: (0, 0)),
        ],
        out_shape=[
            jax.ShapeDtypeStruct((B, 3), jnp.float32),
            jax.ShapeDtypeStruct((B, 1), jnp.int32),
            jax.ShapeDtypeStruct((1, 1), jnp.float32),
        ],
        scratch_shapes=[
            pltpu.VMEM((B, P), jnp.float32),
            pltpu.VMEM((1, P), jnp.float32),
            pltpu.VMEM((1, P), jnp.float32),
        ],
    )(fm, centroids_x, centroids_y, lab3, embeddings, meansT, me)

    return (dist.reshape(B, 3, 1), val.reshape(B), loss[0, 0])
